# baseline (speedup 1.0000x reference)
import functools

import jax
import jax.numpy as jnp
from jax.experimental import pallas as pl
from jax.experimental.pallas import tpu as pltpu


def _net1_body(x_ref, w1_ref, b1_ref, w2_ref, b2_ref, w3_ref, b3_ref,
               w4_ref, b4_ref, w5_ref, b5_ref, o_ref):
    bt = x_ref.shape[3]
    f32 = jnp.float32
    bf16 = jnp.bfloat16

    x = x_ref[...].reshape(28, 28, bt)
    xp = jnp.concatenate([x, jnp.zeros((28, 4, bt), bf16)], axis=1)
    xp = jnp.concatenate([xp, jnp.zeros((3, 32, bt), bf16)], axis=0)

    p1 = jnp.concatenate(
        [xp[4 * ar + 2 * pr:4 * ar + 2 * pr + 5].reshape(160, bt)
         for pr in range(2) for ar in range(7)], axis=-1)

    h1 = jnp.dot(w1_ref[...], p1,
                 preferred_element_type=f32).astype(bf16)
    b1v = b1_ref[...].astype(bf16)
    y1 = []
    for ps in range(2):
        o = ps * 512
        m = jnp.maximum(jnp.maximum(h1[o:o + 128], h1[o + 128:o + 256]),
                        jnp.maximum(h1[o + 256:o + 384], h1[o + 384:o + 512]))
        y1.append(jnp.maximum(m + b1v, 0.0))

    pieces = []
    for Dr in range(5):
        t = Dr // 2
        pr = Dr % 2
        for ps in range(2):
            o = pr * 7 * bt + t * bt
            pieces.append(y1[ps][:, o:o + 5 * bt])
    p2 = jnp.concatenate(pieces, axis=0)

    h2 = jnp.dot(w2_ref[...], p2, preferred_element_type=f32)
    h2 = jnp.maximum(h2 + b2_ref[...], 0.0).astype(bf16)
    y2a = (h2[0:128] + h2[128:256]) + (h2[256:384] + h2[384:512])
    y2b = (h2[512:544] + h2[544:576]) + (h2[576:608] + h2[608:640])
    y2 = jnp.float32(0.25).astype(bf16) * jnp.concatenate([y2a, y2b], axis=0)

    p3 = jnp.concatenate([y2[:, R * bt:(R + 1) * bt] for R in range(5)],
                         axis=0)
    h3 = jnp.dot(w3_ref[...], p3, preferred_element_type=f32)
    h3 = jnp.maximum(h3 + b3_ref[...], 0.0).astype(bf16)

    h4 = jnp.tanh(jnp.dot(w4_ref[...], h3, preferred_element_type=f32)
                  + b4_ref[...]).astype(bf16)
    lg = jnp.dot(w5_ref[...], h4, preferred_element_type=f32) + b5_ref[...]
    m = jnp.max(lg, axis=0, keepdims=True)
    e = jnp.exp(lg - m)
    s = jnp.sum(e, axis=0, keepdims=True)
    o_ref[...] = jnp.transpose(e / s)[None]


@functools.partial(jax.jit, static_argnames=("block_b",))
def _net1_forward(x_nchw, w1, b1, w2, b2, w3, b3, w4, b4, w5, b5, block_b=128):
    B = x_nchw.shape[0]
    nb = B // block_b
    xt = jnp.transpose(x_nchw, (2, 3, 1, 0)).astype(jnp.bfloat16)

    w1r = w1.reshape(5, 32, 512)
    W1p = jnp.stack([w1r, jnp.pad(w1r[:, :30], ((0, 0), (2, 0), (0, 0)))],
                    axis=0)
    W1p = jnp.transpose(W1p, (0, 3, 1, 2)).reshape(1024, 160)
    W1p = W1p.astype(jnp.bfloat16)

    w2r = w2.reshape(1280, 4, 160)
    w2p = jnp.concatenate([w2r[:, g, :128] for g in range(4)]
                          + [w2r[:, g, 128:] for g in range(4)], axis=-1)
    W2p = jnp.transpose(w2p).astype(jnp.bfloat16)
    b2r = b2.reshape(1, 4, 160)
    b2p = jnp.concatenate([b2r[:, g, :128] for g in range(4)]
                          + [b2r[:, g, 128:] for g in range(4)], axis=-1)
    B2p = jnp.transpose(b2p)
    W3t = jnp.transpose(w3).astype(jnp.bfloat16)
    W4t = jnp.transpose(w4).astype(jnp.bfloat16)
    W5t = jnp.transpose(w5).astype(jnp.bfloat16)
    b1t = jnp.transpose(b1)
    b3t = jnp.transpose(b3)
    b4t = jnp.transpose(b4)
    b5t = jnp.transpose(b5)

    out = pl.pallas_call(
        _net1_body,
        out_shape=jax.ShapeDtypeStruct((nb, block_b, 128), jnp.float32),
        grid=(nb,),
        in_specs=[
            pl.BlockSpec((28, 28, 1, block_b), lambda i: (0, 0, 0, i)),
            pl.BlockSpec((1024, 160), lambda i: (0, 0)),
            pl.BlockSpec((128, 1), lambda i: (0, 0)),
            pl.BlockSpec((640, 1280), lambda i: (0, 0)),
            pl.BlockSpec((640, 1), lambda i: (0, 0)),
            pl.BlockSpec((64, 800), lambda i: (0, 0)),
            pl.BlockSpec((64, 1), lambda i: (0, 0)),
            pl.BlockSpec((32, 64), lambda i: (0, 0)),
            pl.BlockSpec((32, 1), lambda i: (0, 0)),
            pl.BlockSpec((128, 32), lambda i: (0, 0)),
            pl.BlockSpec((128, 1), lambda i: (0, 0)),
        ],
        out_specs=pl.BlockSpec((1, block_b, 128), lambda i: (i, 0, 0)),
        compiler_params=pltpu.CompilerParams(
            dimension_semantics=("parallel",),
            vmem_limit_bytes=100 * 1024 * 1024),
    )(xt, W1p, b1t, W2p, B2p, W3t, b3t, W4t, b4t, W5t, b5t)
    return out.reshape(B, 128)[:, :10]


def kernel(x_nchw, w1, b1, w2, b2, w3, b3, w4, b4, w5, b5):
    B = x_nchw.shape[0]
    block_b = 1024 if B % 1024 == 0 else (32 if B % 32 == 0 else 1)
    return _net1_forward(x_nchw, w1, b1, w2, b2, w3, b3, w4, b4, w5, b5,
                         block_b=block_b)


# speedup vs baseline: 1.1969x; 1.1969x over previous
import functools

import jax
import jax.numpy as jnp
from jax.experimental import pallas as pl
from jax.experimental.pallas import tpu as pltpu


def _net1_body(x_ref, w1_ref, b1_ref, w2_ref, b2_ref, w3_ref, b3_ref,
               w4_ref, b4_ref, w5_ref, b5_ref, o_ref):
    bt = x_ref.shape[3]
    f32 = jnp.float32
    bf16 = jnp.bfloat16

    x = x_ref[...].reshape(28, 28, bt).astype(bf16)
    xp = jnp.concatenate([x, jnp.zeros((28, 4, bt), bf16)], axis=1)
    xp = jnp.concatenate([xp, jnp.zeros((3, 32, bt), bf16)], axis=0)

    p1 = jnp.concatenate(
        [xp[4 * ar + 2 * pr:4 * ar + 2 * pr + 5].reshape(160, bt)
         for pr in range(2) for ar in range(7)], axis=-1)

    h1 = jnp.dot(w1_ref[...], p1,
                 preferred_element_type=f32).astype(bf16)
    b1v = b1_ref[...].astype(bf16)
    y1 = []
    for ps in range(2):
        o = ps * 512
        m = jnp.maximum(jnp.maximum(h1[o:o + 128], h1[o + 128:o + 256]),
                        jnp.maximum(h1[o + 256:o + 384], h1[o + 384:o + 512]))
        y1.append(jnp.maximum(m + b1v, 0.0))

    pieces = []
    for Dr in range(5):
        t = Dr // 2
        pr = Dr % 2
        for ps in range(2):
            o = pr * 7 * bt + t * bt
            pieces.append(y1[ps][:, o:o + 5 * bt])
    p2 = jnp.concatenate(pieces, axis=0)

    h2 = jnp.dot(w2_ref[...], p2, preferred_element_type=f32)
    h2 = jnp.maximum(h2 + b2_ref[...], 0.0).astype(bf16)
    y2a = (h2[0:128] + h2[128:256]) + (h2[256:384] + h2[384:512])
    y2b = (h2[512:544] + h2[544:576]) + (h2[576:608] + h2[608:640])
    y2 = jnp.float32(0.25).astype(bf16) * jnp.concatenate([y2a, y2b], axis=0)

    p3 = jnp.concatenate([y2[:, R * bt:(R + 1) * bt] for R in range(5)],
                         axis=0)
    h3 = jnp.dot(w3_ref[...], p3, preferred_element_type=f32)
    h3 = jnp.maximum(h3 + b3_ref[...], 0.0).astype(bf16)

    h4 = jnp.tanh(jnp.dot(w4_ref[...], h3, preferred_element_type=f32)
                  + b4_ref[...]).astype(bf16)
    lg = jnp.dot(w5_ref[...], h4, preferred_element_type=f32) + b5_ref[...]
    m = jnp.max(lg, axis=0, keepdims=True)
    e = jnp.exp(lg - m)
    s = jnp.sum(e, axis=0, keepdims=True)
    o_ref[...] = jnp.transpose(e / s)[None]


@functools.partial(jax.jit, static_argnames=("block_b",))
def _net1_forward(x_nchw, w1, b1, w2, b2, w3, b3, w4, b4, w5, b5, block_b=128):
    B = x_nchw.shape[0]
    nb = B // block_b
    xt = jnp.transpose(x_nchw, (2, 3, 1, 0))

    w1r = w1.reshape(5, 32, 512)
    W1p = jnp.stack([w1r, jnp.pad(w1r[:, :30], ((0, 0), (2, 0), (0, 0)))],
                    axis=0)
    W1p = jnp.transpose(W1p, (0, 3, 1, 2)).reshape(1024, 160)
    W1p = W1p.astype(jnp.bfloat16)

    w2r = w2.reshape(1280, 4, 160)
    w2p = jnp.concatenate([w2r[:, g, :128] for g in range(4)]
                          + [w2r[:, g, 128:] for g in range(4)], axis=-1)
    W2p = jnp.transpose(w2p).astype(jnp.bfloat16)
    b2r = b2.reshape(1, 4, 160)
    b2p = jnp.concatenate([b2r[:, g, :128] for g in range(4)]
                          + [b2r[:, g, 128:] for g in range(4)], axis=-1)
    B2p = jnp.transpose(b2p)
    W3t = jnp.transpose(w3).astype(jnp.bfloat16)
    W4t = jnp.transpose(w4).astype(jnp.bfloat16)
    W5t = jnp.transpose(w5).astype(jnp.bfloat16)
    b1t = jnp.transpose(b1)
    b3t = jnp.transpose(b3)
    b4t = jnp.transpose(b4)
    b5t = jnp.transpose(b5)

    out = pl.pallas_call(
        _net1_body,
        out_shape=jax.ShapeDtypeStruct((nb, block_b, 128), jnp.float32),
        grid=(nb,),
        in_specs=[
            pl.BlockSpec((28, 28, 1, block_b), lambda i: (0, 0, 0, i)),
            pl.BlockSpec((1024, 160), lambda i: (0, 0)),
            pl.BlockSpec((128, 1), lambda i: (0, 0)),
            pl.BlockSpec((640, 1280), lambda i: (0, 0)),
            pl.BlockSpec((640, 1), lambda i: (0, 0)),
            pl.BlockSpec((64, 800), lambda i: (0, 0)),
            pl.BlockSpec((64, 1), lambda i: (0, 0)),
            pl.BlockSpec((32, 64), lambda i: (0, 0)),
            pl.BlockSpec((32, 1), lambda i: (0, 0)),
            pl.BlockSpec((128, 32), lambda i: (0, 0)),
            pl.BlockSpec((128, 1), lambda i: (0, 0)),
        ],
        out_specs=pl.BlockSpec((1, block_b, 128), lambda i: (i, 0, 0)),
        compiler_params=pltpu.CompilerParams(
            dimension_semantics=("parallel",),
            vmem_limit_bytes=100 * 1024 * 1024),
    )(xt, W1p, b1t, W2p, B2p, W3t, b3t, W4t, b4t, W5t, b5t)
    return out.reshape(B, 128)[:, :10]


def kernel(x_nchw, w1, b1, w2, b2, w3, b3, w4, b4, w5, b5):
    B = x_nchw.shape[0]
    block_b = 1024 if B % 1024 == 0 else (32 if B % 32 == 0 else 1)
    return _net1_forward(x_nchw, w1, b1, w2, b2, w3, b3, w4, b4, w5, b5,
                         block_b=block_b)
